# Initial kernel scaffold; baseline (speedup 1.0000x reference)
import jax
import jax.numpy as jnp
from jax.experimental import pallas as pl


def kernel(z, codebook):
    raise NotImplementedError("write your pallas kernel here")


# speedup vs baseline: 1.0866x; 1.0866x over previous
import functools

import jax
import jax.numpy as jnp
from jax import lax
from jax.experimental import pallas as pl
from jax.experimental.pallas import tpu as pltpu
from jax.experimental.pallas import tpu_sc as plsc

N_EMB = 8192
DIM = 32
BM = 512
BK = 1024

NC = 2
NS = 16
NW = NC * NS
ROWS_PER_W = N_EMB // NW
HIST_SLICE = N_EMB // NS


def _argmin_body(z_ref, cb_ref, idx_ref, sum_ref, best_ref):
    i = pl.program_id(0)
    j = pl.program_id(1)
    nj = pl.num_programs(1)

    zb = z_ref[...]
    cb = cb_ref[...]
    a = jnp.sum(zb * zb, axis=1, keepdims=True)
    b = jnp.sum(cb * cb, axis=1)
    m = jax.lax.dot_general(zb, cb, (((1,), (1,)), ((), ())))
    d = (a + b[None, :]) - 2.0 * m

    cmin = jnp.min(d, axis=1, keepdims=True)
    cols = jax.lax.broadcasted_iota(jnp.int32, (BM, BK), 1)
    cand = jnp.min(jnp.where(d == cmin, cols, jnp.int32(2**30)),
                   axis=1, keepdims=True) + j * BK

    @pl.when(j == 0)
    def _():
        best_ref[...] = cmin
        idx_ref[...] = cand

    @pl.when(j > 0)
    def _():
        upd = cmin < best_ref[...]
        best_ref[...] = jnp.where(upd, cmin, best_ref[...])
        idx_ref[...] = jnp.where(upd, cand, idx_ref[...])

    @pl.when(jnp.logical_and(i == 0, j == 0))
    def _():
        sum_ref[0, 0] = 0.0

    @pl.when(j == nj - 1)
    def _():
        sum_ref[0, 0] += jnp.sum(best_ref[...])


def _argmin_call(z_flat, codebook):
    ni = z_flat.shape[0] // BM
    nj = N_EMB // BK
    return pl.pallas_call(
        _argmin_body,
        grid=(ni, nj),
        in_specs=[
            pl.BlockSpec((BM, DIM), lambda i, j: (i, 0)),
            pl.BlockSpec((BK, DIM), lambda i, j: (j, 0)),
        ],
        out_specs=[
            pl.BlockSpec((BM, 1), lambda i, j: (i, 0)),
            pl.BlockSpec(memory_space=pltpu.SMEM),
        ],
        out_shape=[
            jax.ShapeDtypeStruct((z_flat.shape[0], 1), jnp.int32),
            jax.ShapeDtypeStruct((1, 1), jnp.float32),
        ],
        scratch_shapes=[pltpu.VMEM((BM, 1), jnp.float32)],
    )(z_flat, codebook)


def _sc_gather_call(codebook, idx, z_flat):
    mesh = plsc.VectorSubcoreMesh(core_axis_name="c", subcore_axis_name="s")

    @functools.partial(
        pl.kernel,
        mesh=mesh,
        compiler_params=pltpu.CompilerParams(use_tc_tiling_on_sc=False),
        out_type=[
            jax.ShapeDtypeStruct((N_EMB, DIM), jnp.float32),
            jax.ShapeDtypeStruct((NC, N_EMB), jnp.float32),
        ],
        scratch_types=[
            pltpu.VMEM((ROWS_PER_W,), jnp.int32),
            pltpu.VMEM((ROWS_PER_W, DIM), jnp.float32),
            pltpu.VMEM((ROWS_PER_W, DIM), jnp.float32),
            pltpu.VMEM((ROWS_PER_W,), jnp.float32),
            pltpu.VMEM((HIST_SLICE,), jnp.float32),
            pltpu.VMEM_SHARED((N_EMB,), jnp.float32),
            pltpu.SemaphoreType.DMA,
        ],
    )
    def k(cb_hbm, idx_hbm, z_hbm, qst_hbm, hist_hbm,
          idx_v, rows_v, z_v, ones_v, zeros_v, hist_sh, sem):
        cid = lax.axis_index("c")
        sid = lax.axis_index("s")
        wid = sid * NC + cid
        base = wid * ROWS_PER_W

        pltpu.sync_copy(idx_hbm.at[pl.ds(base, ROWS_PER_W)], idx_v)
        pltpu.async_copy(cb_hbm.at[idx_v], rows_v, sem).wait()
        pltpu.sync_copy(z_hbm.at[pl.ds(base, ROWS_PER_W)], z_v)

        @pl.loop(0, ROWS_PER_W // 16)
        def _(t):
            ones_v[pl.ds(t * 16, 16)] = jnp.full((16,), 1.0, jnp.float32)

        @pl.loop(0, HIST_SLICE // 16)
        def _(t):
            zeros_v[pl.ds(t * 16, 16)] = jnp.zeros((16,), jnp.float32)

        pltpu.sync_copy(zeros_v, hist_sh.at[pl.ds(sid * HIST_SLICE,
                                                  HIST_SLICE)])
        plsc.subcore_barrier()
        pltpu.sync_copy(ones_v, hist_sh.at[idx_v], add=True)

        @pl.loop(0, ROWS_PER_W)
        def _(r):
            for h in range(DIM // 16):
                sl = pl.ds(h * 16, 16)
                q = rows_v[r, sl]
                zz = z_v[r, sl]
                rows_v[r, sl] = zz + (q - zz)

        pltpu.sync_copy(rows_v, qst_hbm.at[pl.ds(base, ROWS_PER_W)])

        plsc.subcore_barrier()

        @pl.when(sid == 0)
        def _():
            pltpu.sync_copy(hist_sh, hist_hbm.at[cid])

    return k(codebook, idx, z_flat)


def _finalize_body(hist_ref, dsum_ref, loss_ref, perp_ref):
    counts = hist_ref[0, :] + hist_ref[1, :]
    p = counts * jnp.float32(1.0 / N_EMB)
    ent = jnp.sum(p * jnp.log(p + jnp.float32(1e-10)))
    perp_ref[0, 0] = jnp.exp(-ent)
    loss_ref[0, 0] = dsum_ref[0, 0] * jnp.float32(1.25 / (N_EMB * DIM))


def _finalize_call(hist, dsum):
    return pl.pallas_call(
        _finalize_body,
        in_specs=[
            pl.BlockSpec((NC, N_EMB), lambda: (0, 0)),
            pl.BlockSpec(memory_space=pltpu.SMEM),
        ],
        out_specs=[
            pl.BlockSpec(memory_space=pltpu.SMEM),
            pl.BlockSpec(memory_space=pltpu.SMEM),
        ],
        out_shape=[
            jax.ShapeDtypeStruct((1, 1), jnp.float32),
            jax.ShapeDtypeStruct((1, 1), jnp.float32),
        ],
    )(hist, dsum)


def kernel(z, codebook):
    B, C, H, W = z.shape
    z_flat = jnp.transpose(z, (0, 2, 3, 1)).reshape(-1, DIM)
    idx2, dsum = _argmin_call(z_flat, codebook)
    idx = idx2.reshape(-1)
    q_st_flat, hist = _sc_gather_call(codebook, idx, z_flat)
    loss2, perp2 = _finalize_call(hist, dsum)
    quantized_st = jnp.transpose(q_st_flat.reshape(B, H, W, C), (0, 3, 1, 2))
    return (quantized_st, loss2[0, 0], perp2[0, 0])


# speedup vs baseline: 1.2974x; 1.1940x over previous
import functools

import jax
import jax.numpy as jnp
from jax import lax
from jax.experimental import pallas as pl
from jax.experimental.pallas import tpu as pltpu
from jax.experimental.pallas import tpu_sc as plsc

N_EMB = 8192
DIM = 32
BM = 512
BK = 2048

NC = 2
NS = 16
NW = NC * NS
ROWS_PER_W = N_EMB // NW
HIST_SLICE = N_EMB // NS


def _argmin_body(z_ref, cb_ref, idx_ref, sum_ref, best_ref):
    i = pl.program_id(0)
    j = pl.program_id(1)
    nj = pl.num_programs(1)

    zb = z_ref[...]
    cbn = cb_ref[...]
    a = jnp.sum(zb * zb, axis=1, keepdims=True)
    b = jnp.sum(cbn * cbn, axis=1) * 0.25
    m = jax.lax.dot_general(zb, cbn, (((1,), (1,)), ((), ())))
    d = (a + b[None, :]) + m

    cmin = jnp.min(d, axis=1, keepdims=True)
    cols = jax.lax.broadcasted_iota(jnp.int32, (BM, BK), 1)
    cand = jnp.min(jnp.where(d == cmin, cols, jnp.int32(2**30)),
                   axis=1, keepdims=True) + j * BK

    @pl.when(j == 0)
    def _():
        best_ref[...] = cmin
        idx_ref[...] = cand

    @pl.when(j > 0)
    def _():
        upd = cmin < best_ref[...]
        best_ref[...] = jnp.where(upd, cmin, best_ref[...])
        idx_ref[...] = jnp.where(upd, cand, idx_ref[...])

    @pl.when(jnp.logical_and(i == 0, j == 0))
    def _():
        sum_ref[0, 0] = 0.0

    @pl.when(j == nj - 1)
    def _():
        sum_ref[0, 0] += jnp.sum(best_ref[...])


def _argmin_call(z_flat, codebook):
    ni = z_flat.shape[0] // BM
    nj = N_EMB // BK
    return pl.pallas_call(
        _argmin_body,
        grid=(ni, nj),
        in_specs=[
            pl.BlockSpec((BM, DIM), lambda i, j: (i, 0)),
            pl.BlockSpec((BK, DIM), lambda i, j: (j, 0)),
        ],
        out_specs=[
            pl.BlockSpec((BM, 1), lambda i, j: (i, 0)),
            pl.BlockSpec(memory_space=pltpu.SMEM),
        ],
        out_shape=[
            jax.ShapeDtypeStruct((z_flat.shape[0], 1), jnp.int32),
            jax.ShapeDtypeStruct((1, 1), jnp.float32),
        ],
        scratch_shapes=[pltpu.VMEM((BM, 1), jnp.float32)],
    )(z_flat, codebook)


def _sc_gather_call(codebook, idx, z_flat):
    mesh = plsc.VectorSubcoreMesh(core_axis_name="c", subcore_axis_name="s")

    @functools.partial(
        pl.kernel,
        mesh=mesh,
        compiler_params=pltpu.CompilerParams(use_tc_tiling_on_sc=False),
        out_type=[
            jax.ShapeDtypeStruct((N_EMB, DIM), jnp.float32),
            jax.ShapeDtypeStruct((NC, N_EMB), jnp.float32),
        ],
        scratch_types=[
            pltpu.VMEM((ROWS_PER_W,), jnp.int32),
            pltpu.VMEM((ROWS_PER_W, DIM), jnp.float32),
            pltpu.VMEM((ROWS_PER_W, DIM), jnp.float32),
            pltpu.VMEM((ROWS_PER_W,), jnp.float32),
            pltpu.VMEM((HIST_SLICE,), jnp.float32),
            pltpu.VMEM_SHARED((N_EMB,), jnp.float32),
            pltpu.SemaphoreType.DMA,
        ],
    )
    def k(cb_hbm, idx_hbm, z_hbm, qst_hbm, hist_hbm,
          idx_v, rows_v, z_v, ones_v, zeros_v, hist_sh, sem):
        cid = lax.axis_index("c")
        sid = lax.axis_index("s")
        wid = sid * NC + cid
        base = wid * ROWS_PER_W

        pltpu.sync_copy(idx_hbm.at[pl.ds(base, ROWS_PER_W)], idx_v)
        pltpu.async_copy(cb_hbm.at[idx_v], rows_v, sem).wait()
        pltpu.sync_copy(z_hbm.at[pl.ds(base, ROWS_PER_W)], z_v)

        @pl.loop(0, ROWS_PER_W // 16)
        def _(t):
            ones_v[pl.ds(t * 16, 16)] = jnp.full((16,), 1.0, jnp.float32)

        @pl.loop(0, HIST_SLICE // 16)
        def _(t):
            zeros_v[pl.ds(t * 16, 16)] = jnp.zeros((16,), jnp.float32)

        pltpu.sync_copy(zeros_v, hist_sh.at[pl.ds(sid * HIST_SLICE,
                                                  HIST_SLICE)])
        plsc.subcore_barrier()
        pltpu.sync_copy(ones_v, hist_sh.at[idx_v], add=True)

        @pl.loop(0, ROWS_PER_W)
        def _(r):
            for h in range(DIM // 16):
                sl = pl.ds(h * 16, 16)
                q = rows_v[r, sl]
                zz = z_v[r, sl]
                rows_v[r, sl] = zz + (q - zz)

        pltpu.sync_copy(rows_v, qst_hbm.at[pl.ds(base, ROWS_PER_W)])

        plsc.subcore_barrier()

        @pl.when(sid == 0)
        def _():
            pltpu.sync_copy(hist_sh, hist_hbm.at[cid])

    return k(codebook, idx, z_flat)


def _finalize_body(hist_ref, dsum_ref, loss_ref, perp_ref):
    counts = hist_ref[0, :] + hist_ref[1, :]
    p = counts * jnp.float32(1.0 / N_EMB)
    ent = jnp.sum(p * jnp.log(p + jnp.float32(1e-10)))
    perp_ref[0, 0] = jnp.exp(-ent)
    loss_ref[0, 0] = dsum_ref[0, 0] * jnp.float32(1.25 / (N_EMB * DIM))


def _finalize_call(hist, dsum):
    return pl.pallas_call(
        _finalize_body,
        in_specs=[
            pl.BlockSpec((NC, N_EMB), lambda: (0, 0)),
            pl.BlockSpec(memory_space=pltpu.SMEM),
        ],
        out_specs=[
            pl.BlockSpec(memory_space=pltpu.SMEM),
            pl.BlockSpec(memory_space=pltpu.SMEM),
        ],
        out_shape=[
            jax.ShapeDtypeStruct((1, 1), jnp.float32),
            jax.ShapeDtypeStruct((1, 1), jnp.float32),
        ],
    )(hist, dsum)


def kernel(z, codebook):
    B, C, H, W = z.shape
    z_flat = jnp.transpose(z, (0, 2, 3, 1)).reshape(-1, DIM)
    idx2, dsum = _argmin_call(z_flat, codebook * jnp.float32(-2.0))
    idx = idx2.reshape(-1)
    q_st_flat, hist = _sc_gather_call(codebook, idx, z_flat)
    loss2, perp2 = _finalize_call(hist, dsum)
    quantized_st = jnp.transpose(q_st_flat.reshape(B, H, W, C), (0, 3, 1, 2))
    return (quantized_st, loss2[0, 0], perp2[0, 0])


# speedup vs baseline: 1.4668x; 1.1306x over previous
import functools

import jax
import jax.numpy as jnp
from jax import lax
from jax.experimental import pallas as pl
from jax.experimental.pallas import tpu as pltpu
from jax.experimental.pallas import tpu_sc as plsc

N_EMB = 8192
DIM = 32
BM = 512
BK = 8192

NC = 2
NS = 16
NW = NC * NS
ROWS_PER_W = N_EMB // NW
HIST_SLICE = N_EMB // NS


def _argmin_body(z_ref, cb_ref, idx_ref, sum_ref, best_ref):
    i = pl.program_id(0)
    j = pl.program_id(1)
    nj = pl.num_programs(1)

    zb = z_ref[...]
    cbn = cb_ref[...]
    a = jnp.sum(zb * zb, axis=1, keepdims=True)
    b = jnp.sum(cbn * cbn, axis=1) * 0.25
    m = jax.lax.dot_general(zb, cbn, (((1,), (1,)), ((), ())))
    d = (a + b[None, :]) + m

    cmin = jnp.min(d, axis=1, keepdims=True)
    cols = jax.lax.broadcasted_iota(jnp.int32, (BM, BK), 1)
    cand = jnp.min(jnp.where(d == cmin, cols, jnp.int32(2**30)),
                   axis=1, keepdims=True) + j * BK

    @pl.when(j == 0)
    def _():
        best_ref[...] = cmin
        idx_ref[...] = cand

    @pl.when(j > 0)
    def _():
        upd = cmin < best_ref[...]
        best_ref[...] = jnp.where(upd, cmin, best_ref[...])
        idx_ref[...] = jnp.where(upd, cand, idx_ref[...])

    @pl.when(jnp.logical_and(i == 0, j == 0))
    def _():
        sum_ref[0, 0] = 0.0

    @pl.when(j == nj - 1)
    def _():
        sum_ref[0, 0] += jnp.sum(best_ref[...])


def _argmin_call(z_flat, codebook):
    ni = z_flat.shape[0] // BM
    nj = N_EMB // BK
    return pl.pallas_call(
        _argmin_body,
        grid=(ni, nj),
        in_specs=[
            pl.BlockSpec((BM, DIM), lambda i, j: (i, 0)),
            pl.BlockSpec((BK, DIM), lambda i, j: (j, 0)),
        ],
        out_specs=[
            pl.BlockSpec((BM, 1), lambda i, j: (i, 0)),
            pl.BlockSpec(memory_space=pltpu.SMEM),
        ],
        out_shape=[
            jax.ShapeDtypeStruct((z_flat.shape[0], 1), jnp.int32),
            jax.ShapeDtypeStruct((1, 1), jnp.float32),
        ],
        scratch_shapes=[pltpu.VMEM((BM, 1), jnp.float32)],
    )(z_flat, codebook)


def _sc_gather_call(codebook, idx, z_flat):
    mesh = plsc.VectorSubcoreMesh(core_axis_name="c", subcore_axis_name="s")

    @functools.partial(
        pl.kernel,
        mesh=mesh,
        compiler_params=pltpu.CompilerParams(use_tc_tiling_on_sc=False),
        out_type=[
            jax.ShapeDtypeStruct((N_EMB, DIM), jnp.float32),
            jax.ShapeDtypeStruct((NC, N_EMB), jnp.float32),
        ],
        scratch_types=[
            pltpu.VMEM((ROWS_PER_W,), jnp.int32),
            pltpu.VMEM((ROWS_PER_W, DIM), jnp.float32),
            pltpu.VMEM((ROWS_PER_W, DIM), jnp.float32),
            pltpu.VMEM((ROWS_PER_W,), jnp.float32),
            pltpu.VMEM((HIST_SLICE,), jnp.float32),
            pltpu.VMEM_SHARED((N_EMB,), jnp.float32),
            pltpu.SemaphoreType.DMA,
        ],
    )
    def k(cb_hbm, idx_hbm, z_hbm, qst_hbm, hist_hbm,
          idx_v, rows_v, z_v, ones_v, zeros_v, hist_sh, sem):
        cid = lax.axis_index("c")
        sid = lax.axis_index("s")
        wid = sid * NC + cid
        base = wid * ROWS_PER_W

        pltpu.sync_copy(idx_hbm.at[pl.ds(base, ROWS_PER_W)], idx_v)
        pltpu.async_copy(cb_hbm.at[idx_v], rows_v, sem).wait()
        pltpu.sync_copy(z_hbm.at[pl.ds(base, ROWS_PER_W)], z_v)

        @pl.loop(0, ROWS_PER_W // 16)
        def _(t):
            ones_v[pl.ds(t * 16, 16)] = jnp.full((16,), 1.0, jnp.float32)

        @pl.loop(0, HIST_SLICE // 16)
        def _(t):
            zeros_v[pl.ds(t * 16, 16)] = jnp.zeros((16,), jnp.float32)

        pltpu.sync_copy(zeros_v, hist_sh.at[pl.ds(sid * HIST_SLICE,
                                                  HIST_SLICE)])
        plsc.subcore_barrier()
        pltpu.sync_copy(ones_v, hist_sh.at[idx_v], add=True)

        @pl.loop(0, ROWS_PER_W)
        def _(r):
            for h in range(DIM // 16):
                sl = pl.ds(h * 16, 16)
                q = rows_v[r, sl]
                zz = z_v[r, sl]
                rows_v[r, sl] = zz + (q - zz)

        pltpu.sync_copy(rows_v, qst_hbm.at[pl.ds(base, ROWS_PER_W)])

        plsc.subcore_barrier()

        @pl.when(sid == 0)
        def _():
            pltpu.sync_copy(hist_sh, hist_hbm.at[cid])

    return k(codebook, idx, z_flat)


def _finalize_body(hist_ref, dsum_ref, loss_ref, perp_ref):
    counts = hist_ref[0, :] + hist_ref[1, :]
    p = counts * jnp.float32(1.0 / N_EMB)
    ent = jnp.sum(p * jnp.log(p + jnp.float32(1e-10)))
    perp_ref[0, 0] = jnp.exp(-ent)
    loss_ref[0, 0] = dsum_ref[0, 0] * jnp.float32(1.25 / (N_EMB * DIM))


def _finalize_call(hist, dsum):
    return pl.pallas_call(
        _finalize_body,
        in_specs=[
            pl.BlockSpec((NC, N_EMB), lambda: (0, 0)),
            pl.BlockSpec(memory_space=pltpu.SMEM),
        ],
        out_specs=[
            pl.BlockSpec(memory_space=pltpu.SMEM),
            pl.BlockSpec(memory_space=pltpu.SMEM),
        ],
        out_shape=[
            jax.ShapeDtypeStruct((1, 1), jnp.float32),
            jax.ShapeDtypeStruct((1, 1), jnp.float32),
        ],
    )(hist, dsum)


def kernel(z, codebook):
    B, C, H, W = z.shape
    z_flat = jnp.transpose(z, (0, 2, 3, 1)).reshape(-1, DIM)
    idx2, dsum = _argmin_call(z_flat, codebook * jnp.float32(-2.0))
    idx = idx2.reshape(-1)
    q_st_flat, hist = _sc_gather_call(codebook, idx, z_flat)
    loss2, perp2 = _finalize_call(hist, dsum)
    quantized_st = jnp.transpose(q_st_flat.reshape(B, H, W, C), (0, 3, 1, 2))
    return (quantized_st, loss2[0, 0], perp2[0, 0])
